# baseline (speedup 1.0000x reference)
import functools

import jax
import jax.numpy as jnp
from jax import lax
from jax.experimental import pallas as pl
from jax.experimental.pallas import tpu as pltpu
from jax.experimental.pallas import tpu_sc as plsc

DIM_HALF = 64
DC = 2 * DIM_HALF
LBASE = 128
EXT = 9216
SEQ = 8192
N_CACHE_BLKS = EXT // LBASE

NC = 2
NS = 16
NW = NC * NS
BPW = SEQ // NW
HALF_BPW = BPW // 2

SEQ_BLK = 1024
N_SEQ_BLKS = SEQ // SEQ_BLK


def _tables_body(invf_ref, a1_ref, a2_ref, l1_ref, l2_ref):
    l = (lax.broadcasted_iota(jnp.int32, (LBASE, DIM_HALF), 0)
         .astype(jnp.float32))
    ang_lo = l * invf_ref[...]
    ang_hi = ang_lo * float(LBASE)
    cl = jnp.cos(ang_lo)
    sl = jnp.sin(ang_lo)
    ch = jnp.cos(ang_hi)
    sh = jnp.sin(ang_hi)
    a1_ref[...] = jnp.concatenate([ch, sh], axis=1)
    a2_ref[...] = jnp.concatenate([sh, ch], axis=1)
    l1_ref[...] = jnp.concatenate([cl, cl], axis=1)
    l2_ref[...] = jnp.concatenate([-sl, sl], axis=1)


def _combine_body(a1_ref, a2_ref, l1_ref, l2_ref, out_ref):
    i = pl.program_id(0)
    a1 = a1_ref[pl.ds(i, 1), :]
    a2 = a2_ref[pl.ds(i, 1), :]
    out_ref[...] = a1 * l1_ref[...] + a2 * l2_ref[...]


def _build_cache(inv_freq):
    invf2d = inv_freq.reshape(1, DIM_HALF)
    tab_shape = jax.ShapeDtypeStruct((LBASE, DC), jnp.float32)
    a1, a2, l1, l2 = pl.pallas_call(
        _tables_body,
        out_shape=[tab_shape, tab_shape, tab_shape, tab_shape],
    )(invf2d)
    full_spec = pl.BlockSpec((LBASE, DC), lambda i: (0, 0))
    return pl.pallas_call(
        _combine_body,
        grid=(N_CACHE_BLKS,),
        in_specs=[full_spec, full_spec, full_spec, full_spec],
        out_specs=pl.BlockSpec((LBASE, DC), lambda i: (i, 0)),
        out_shape=jax.ShapeDtypeStruct((EXT, DC), jnp.float32),
    )(a1, a2, l1, l2)


@functools.cache
def _make_sc_gather():
    mesh = plsc.VectorSubcoreMesh(core_axis_name="c", subcore_axis_name="s")

    @functools.partial(
        pl.kernel,
        mesh=mesh,
        out_type=jax.ShapeDtypeStruct((SEQ, DC), jnp.float32),
        scratch_types=[
            pltpu.VMEM((HALF_BPW,), jnp.int32),
            pltpu.VMEM((HALF_BPW,), jnp.int32),
            pltpu.VMEM((HALF_BPW, DC), jnp.float32),
            pltpu.VMEM((HALF_BPW, DC), jnp.float32),
            pltpu.SemaphoreType.DMA,
            pltpu.SemaphoreType.DMA,
        ],
    )
    def _sc_gather(cache_hbm, pos_hbm, out_hbm,
                   idx_a, idx_b, rows_a, rows_b, sem_a, sem_b):
        wid = lax.axis_index("s") * NC + lax.axis_index("c")
        base = wid * BPW
        pltpu.sync_copy(pos_hbm.at[pl.ds(base, HALF_BPW)], idx_a)
        pltpu.sync_copy(pos_hbm.at[pl.ds(base + HALF_BPW, HALF_BPW)], idx_b)
        cp_a = pltpu.async_copy(cache_hbm.at[idx_a], rows_a, sem_a)
        cp_b = pltpu.async_copy(cache_hbm.at[idx_b], rows_b, sem_b)
        cp_a.wait()
        pltpu.sync_copy(rows_a, out_hbm.at[pl.ds(base, HALF_BPW)])
        cp_b.wait()
        pltpu.sync_copy(rows_b, out_hbm.at[pl.ds(base + HALF_BPW, HALF_BPW)])

    return _sc_gather


def _split_body(both_ref, cos_ref, sin_ref):
    bt = both_ref[...].T
    cos_ref[...] = bt[0:DIM_HALF, :]
    sin_ref[...] = bt[DIM_HALF:DC, :]


def _split_transpose(both):
    return pl.pallas_call(
        _split_body,
        grid=(N_SEQ_BLKS,),
        in_specs=[pl.BlockSpec((SEQ_BLK, DC), lambda i: (i, 0))],
        out_specs=[
            pl.BlockSpec((DIM_HALF, SEQ_BLK), lambda i: (0, i)),
            pl.BlockSpec((DIM_HALF, SEQ_BLK), lambda i: (0, i)),
        ],
        out_shape=[
            jax.ShapeDtypeStruct((DIM_HALF, SEQ), jnp.float32),
            jax.ShapeDtypeStruct((DIM_HALF, SEQ), jnp.float32),
        ],
    )(both)


def kernel(positions, inv_freq):
    cache = _build_cache(inv_freq)
    pos32 = positions.astype(jnp.int32)
    both = _make_sc_gather()(cache, pos32)
    cos_t, sin_t = _split_transpose(both)
    return (cos_t.T, sin_t.T)


# speedup vs baseline: 1.5297x; 1.5297x over previous
import functools

import jax
import jax.numpy as jnp
from jax import lax
from jax.experimental import pallas as pl
from jax.experimental.pallas import tpu as pltpu
from jax.experimental.pallas import tpu_sc as plsc

DIM_HALF = 64
DC = 2 * DIM_HALF
LBASE = 128
EXT = 9216
SEQ = 8192
HPB = 8
CACHE_BLK = HPB * LBASE
N_BIG_BLKS = EXT // CACHE_BLK

NC = 2
NS = 16
NW = NC * NS
BPW = SEQ // NW
HALF_BPW = BPW // 2

SEQ_BLK = 1024
N_SEQ_BLKS = SEQ // SEQ_BLK


def _tables_body(invf_ref, a1_ref, a2_ref, l1_ref, l2_ref):
    l = (lax.broadcasted_iota(jnp.int32, (LBASE, DIM_HALF), 0)
         .astype(jnp.float32))
    ang_lo = l * invf_ref[...]
    ang_hi = ang_lo * float(LBASE)
    cl = jnp.cos(ang_lo)
    sl = jnp.sin(ang_lo)
    ch = jnp.cos(ang_hi)
    sh = jnp.sin(ang_hi)
    a1_ref[...] = jnp.concatenate([ch, sh], axis=1)
    a2_ref[...] = jnp.concatenate([sh, ch], axis=1)
    l1_ref[...] = jnp.concatenate([cl, cl], axis=1)
    l2_ref[...] = jnp.concatenate([-sl, sl], axis=1)


def _combine_body(a1_ref, a2_ref, l1_ref, l2_ref, out_ref):
    i = pl.program_id(0)
    a1 = a1_ref[pl.ds(HPB * i, HPB), :].reshape(HPB, 1, DC)
    a2 = a2_ref[pl.ds(HPB * i, HPB), :].reshape(HPB, 1, DC)
    l1 = l1_ref[...].reshape(1, LBASE, DC)
    l2 = l2_ref[...].reshape(1, LBASE, DC)
    out = a1 * l1 + a2 * l2
    out_ref[...] = out.reshape(CACHE_BLK, DC)


def _build_cache(inv_freq):
    invf2d = inv_freq.reshape(1, DIM_HALF)
    tab_shape = jax.ShapeDtypeStruct((LBASE, DC), jnp.float32)
    a1, a2, l1, l2 = pl.pallas_call(
        _tables_body,
        out_shape=[tab_shape, tab_shape, tab_shape, tab_shape],
    )(invf2d)
    full_spec = pl.BlockSpec((LBASE, DC), lambda i: (0, 0))
    return pl.pallas_call(
        _combine_body,
        grid=(N_BIG_BLKS,),
        in_specs=[full_spec, full_spec, full_spec, full_spec],
        out_specs=pl.BlockSpec((CACHE_BLK, DC), lambda i: (i, 0)),
        out_shape=jax.ShapeDtypeStruct((EXT, DC), jnp.float32),
    )(a1, a2, l1, l2)


@functools.cache
def _make_sc_gather():
    mesh = plsc.VectorSubcoreMesh(core_axis_name="c", subcore_axis_name="s")

    @functools.partial(
        pl.kernel,
        mesh=mesh,
        out_type=jax.ShapeDtypeStruct((SEQ, DC), jnp.float32),
        scratch_types=[
            pltpu.VMEM((BPW,), jnp.int32),
            pltpu.VMEM((BPW, DC), jnp.float32),
            pltpu.SemaphoreType.DMA,
        ],
    )
    def _sc_gather(cache_hbm, pos_hbm, out_hbm, idx_v, rows_v, sem):
        wid = lax.axis_index("s") * NC + lax.axis_index("c")
        base = wid * BPW
        pltpu.sync_copy(pos_hbm.at[pl.ds(base, BPW)], idx_v)
        pltpu.async_copy(cache_hbm.at[idx_v], rows_v, sem).wait()
        pltpu.sync_copy(rows_v, out_hbm.at[pl.ds(base, BPW)])

    return _sc_gather


def _split_body(both_ref, cos_ref, sin_ref):
    bt = both_ref[...].T
    cos_ref[...] = bt[0:DIM_HALF, :]
    sin_ref[...] = bt[DIM_HALF:DC, :]


def _split_transpose(both):
    return pl.pallas_call(
        _split_body,
        grid=(N_SEQ_BLKS,),
        in_specs=[pl.BlockSpec((SEQ_BLK, DC), lambda i: (i, 0))],
        out_specs=[
            pl.BlockSpec((DIM_HALF, SEQ_BLK), lambda i: (0, i)),
            pl.BlockSpec((DIM_HALF, SEQ_BLK), lambda i: (0, i)),
        ],
        out_shape=[
            jax.ShapeDtypeStruct((DIM_HALF, SEQ), jnp.float32),
            jax.ShapeDtypeStruct((DIM_HALF, SEQ), jnp.float32),
        ],
    )(both)


def kernel(positions, inv_freq):
    cache = _build_cache(inv_freq)
    pos32 = positions.astype(jnp.int32)
    both = _make_sc_gather()(cache, pos32)
    cos_t, sin_t = _split_transpose(both)
    return (cos_t.T, sin_t.T)


# speedup vs baseline: 1.6203x; 1.0592x over previous
import functools

import jax
import jax.numpy as jnp
from jax import lax
from jax.experimental import pallas as pl
from jax.experimental.pallas import tpu as pltpu
from jax.experimental.pallas import tpu_sc as plsc

DIM_HALF = 64
DC = 2 * DIM_HALF
LBASE = 128
EXT = 9216
SEQ = 8192
HPB = 8
CACHE_BLK = HPB * LBASE
N_BIG_BLKS = EXT // CACHE_BLK

NC = 2
NS = 16
NW = NC * NS
BPW = SEQ // NW
HALF_BPW = BPW // 2

SEQ_BLK = 2048
N_SEQ_BLKS = SEQ // SEQ_BLK


def _tables_body(invf_ref, a1_ref, a2_ref, l1_ref, l2_ref):
    l = (lax.broadcasted_iota(jnp.int32, (LBASE, DIM_HALF), 0)
         .astype(jnp.float32))
    ang_lo = l * invf_ref[...]
    ang_hi = ang_lo * float(LBASE)
    cl = jnp.cos(ang_lo)
    sl = jnp.sin(ang_lo)
    ch = jnp.cos(ang_hi)
    sh = jnp.sin(ang_hi)
    a1_ref[...] = jnp.concatenate([ch, sh], axis=1)
    a2_ref[...] = jnp.concatenate([sh, ch], axis=1)
    l1_ref[...] = jnp.concatenate([cl, cl], axis=1)
    l2_ref[...] = jnp.concatenate([-sl, sl], axis=1)


def _combine_body(a1_ref, a2_ref, l1_ref, l2_ref, out_ref):
    i = pl.program_id(0)
    a1 = a1_ref[pl.ds(HPB * i, HPB), :].reshape(HPB, 1, DC)
    a2 = a2_ref[pl.ds(HPB * i, HPB), :].reshape(HPB, 1, DC)
    l1 = l1_ref[...].reshape(1, LBASE, DC)
    l2 = l2_ref[...].reshape(1, LBASE, DC)
    out = a1 * l1 + a2 * l2
    out_ref[...] = out.reshape(CACHE_BLK, DC)


def _build_cache(inv_freq):
    invf2d = inv_freq.reshape(1, DIM_HALF)
    tab_shape = jax.ShapeDtypeStruct((LBASE, DC), jnp.float32)
    a1, a2, l1, l2 = pl.pallas_call(
        _tables_body,
        out_shape=[tab_shape, tab_shape, tab_shape, tab_shape],
    )(invf2d)
    full_spec = pl.BlockSpec((LBASE, DC), lambda i: (0, 0))
    return pl.pallas_call(
        _combine_body,
        grid=(N_BIG_BLKS,),
        in_specs=[full_spec, full_spec, full_spec, full_spec],
        out_specs=pl.BlockSpec((CACHE_BLK, DC), lambda i: (i, 0)),
        out_shape=jax.ShapeDtypeStruct((EXT, DC), jnp.float32),
    )(a1, a2, l1, l2)


@functools.cache
def _make_sc_gather():
    mesh = plsc.VectorSubcoreMesh(core_axis_name="c", subcore_axis_name="s")

    @functools.partial(
        pl.kernel,
        mesh=mesh,
        out_type=jax.ShapeDtypeStruct((SEQ, DC), jnp.float32),
        scratch_types=[
            pltpu.VMEM((BPW,), jnp.int32),
            pltpu.VMEM((BPW, DC), jnp.float32),
            pltpu.SemaphoreType.DMA,
        ],
    )
    def _sc_gather(cache_hbm, pos_hbm, out_hbm, idx_v, rows_v, sem):
        wid = lax.axis_index("s") * NC + lax.axis_index("c")
        base = wid * BPW
        pltpu.sync_copy(pos_hbm.at[pl.ds(base, BPW)], idx_v)
        pltpu.async_copy(cache_hbm.at[idx_v], rows_v, sem).wait()
        pltpu.sync_copy(rows_v, out_hbm.at[pl.ds(base, BPW)])

    return _sc_gather


def _split_body(both_ref, cos_ref, sin_ref):
    bt = both_ref[...].T
    cos_ref[...] = bt[0:DIM_HALF, :]
    sin_ref[...] = bt[DIM_HALF:DC, :]


def _split_transpose(both):
    return pl.pallas_call(
        _split_body,
        grid=(N_SEQ_BLKS,),
        in_specs=[pl.BlockSpec((SEQ_BLK, DC), lambda i: (i, 0))],
        out_specs=[
            pl.BlockSpec((DIM_HALF, SEQ_BLK), lambda i: (0, i)),
            pl.BlockSpec((DIM_HALF, SEQ_BLK), lambda i: (0, i)),
        ],
        out_shape=[
            jax.ShapeDtypeStruct((DIM_HALF, SEQ), jnp.float32),
            jax.ShapeDtypeStruct((DIM_HALF, SEQ), jnp.float32),
        ],
    )(both)


def kernel(positions, inv_freq):
    cache = _build_cache(inv_freq)
    pos32 = positions.astype(jnp.int32)
    both = _make_sc_gather()(cache, pos32)
    cos_t, sin_t = _split_transpose(both)
    return (cos_t.T, sin_t.T)


# speedup vs baseline: 1.6819x; 1.0380x over previous
import functools

import jax
import jax.numpy as jnp
from jax import lax
from jax.experimental import pallas as pl
from jax.experimental.pallas import tpu as pltpu
from jax.experimental.pallas import tpu_sc as plsc

DIM_HALF = 64
DC = 2 * DIM_HALF
LBASE = 128
EXT = 9216
SEQ = 8192
HPB = 8
CACHE_BLK = HPB * LBASE
N_BIG_BLKS = EXT // CACHE_BLK

NC = 2
NS = 16
NW = NC * NS
BPW = SEQ // NW
HALF_BPW = BPW // 2

SEQ_BLK = 4096
N_SEQ_BLKS = SEQ // SEQ_BLK


def _tables_body(invf_ref, a1_ref, a2_ref, l1_ref, l2_ref):
    l = (lax.broadcasted_iota(jnp.int32, (LBASE, DIM_HALF), 0)
         .astype(jnp.float32))
    ang_lo = l * invf_ref[...]
    ang_hi = ang_lo * float(LBASE)
    cl = jnp.cos(ang_lo)
    sl = jnp.sin(ang_lo)
    ch = jnp.cos(ang_hi)
    sh = jnp.sin(ang_hi)
    a1_ref[...] = jnp.concatenate([ch, sh], axis=1)
    a2_ref[...] = jnp.concatenate([sh, ch], axis=1)
    l1_ref[...] = jnp.concatenate([cl, cl], axis=1)
    l2_ref[...] = jnp.concatenate([-sl, sl], axis=1)


def _combine_body(a1_ref, a2_ref, l1_ref, l2_ref, out_ref):
    i = pl.program_id(0)
    a1 = a1_ref[pl.ds(HPB * i, HPB), :].reshape(HPB, 1, DC)
    a2 = a2_ref[pl.ds(HPB * i, HPB), :].reshape(HPB, 1, DC)
    l1 = l1_ref[...].reshape(1, LBASE, DC)
    l2 = l2_ref[...].reshape(1, LBASE, DC)
    out = a1 * l1 + a2 * l2
    out_ref[...] = out.reshape(CACHE_BLK, DC)


def _build_cache(inv_freq):
    invf2d = inv_freq.reshape(1, DIM_HALF)
    tab_shape = jax.ShapeDtypeStruct((LBASE, DC), jnp.float32)
    a1, a2, l1, l2 = pl.pallas_call(
        _tables_body,
        out_shape=[tab_shape, tab_shape, tab_shape, tab_shape],
    )(invf2d)
    full_spec = pl.BlockSpec((LBASE, DC), lambda i: (0, 0))
    return pl.pallas_call(
        _combine_body,
        grid=(N_BIG_BLKS,),
        in_specs=[full_spec, full_spec, full_spec, full_spec],
        out_specs=pl.BlockSpec((CACHE_BLK, DC), lambda i: (i, 0)),
        out_shape=jax.ShapeDtypeStruct((EXT, DC), jnp.float32),
    )(a1, a2, l1, l2)


@functools.cache
def _make_sc_gather():
    mesh = plsc.VectorSubcoreMesh(core_axis_name="c", subcore_axis_name="s")

    @functools.partial(
        pl.kernel,
        mesh=mesh,
        out_type=jax.ShapeDtypeStruct((SEQ, DC), jnp.float32),
        scratch_types=[
            pltpu.VMEM((BPW,), jnp.int32),
            pltpu.VMEM((BPW, DC), jnp.float32),
            pltpu.SemaphoreType.DMA,
        ],
    )
    def _sc_gather(cache_hbm, pos_hbm, out_hbm, idx_v, rows_v, sem):
        wid = lax.axis_index("s") * NC + lax.axis_index("c")
        base = wid * BPW
        pltpu.sync_copy(pos_hbm.at[pl.ds(base, BPW)], idx_v)
        pltpu.async_copy(cache_hbm.at[idx_v], rows_v, sem).wait()
        pltpu.sync_copy(rows_v, out_hbm.at[pl.ds(base, BPW)])

    return _sc_gather


def _split_body(both_ref, cos_ref, sin_ref):
    bt = both_ref[...].T
    cos_ref[...] = bt[0:DIM_HALF, :]
    sin_ref[...] = bt[DIM_HALF:DC, :]


def _split_transpose(both):
    return pl.pallas_call(
        _split_body,
        grid=(N_SEQ_BLKS,),
        in_specs=[pl.BlockSpec((SEQ_BLK, DC), lambda i: (i, 0))],
        out_specs=[
            pl.BlockSpec((DIM_HALF, SEQ_BLK), lambda i: (0, i)),
            pl.BlockSpec((DIM_HALF, SEQ_BLK), lambda i: (0, i)),
        ],
        out_shape=[
            jax.ShapeDtypeStruct((DIM_HALF, SEQ), jnp.float32),
            jax.ShapeDtypeStruct((DIM_HALF, SEQ), jnp.float32),
        ],
    )(both)


def kernel(positions, inv_freq):
    cache = _build_cache(inv_freq)
    pos32 = positions.astype(jnp.int32)
    both = _make_sc_gather()(cache, pos32)
    cos_t, sin_t = _split_transpose(both)
    return (cos_t.T, sin_t.T)
